# baseline (speedup 1.0000x reference)
import jax
import jax.numpy as jnp
from jax.experimental import pallas as pl
from jax.experimental.pallas import tpu as pltpu

B, L, D = 256, 512, 128
BB = 16
CH = 128


def _kernel(s_ref, d_ref, w1c_ref, b1c_ref, w2_ref, b2_ref, out_s_ref, out_d_ref):
    w1_col = w1c_ref[...]
    b1_col = b1c_ref[...]
    w2b = w2_ref[...]
    b2x2 = b2_ref[...]

    def counts(query, corpus_col):
        qb = jnp.broadcast_to(query, (CH, L))
        acc = None
        for c in range(0, L, CH):
            corp = jnp.broadcast_to(corpus_col[c:c + CH], (CH, L))
            x = jnp.where(qb == corp, jnp.int16(1), jnp.int16(0))
            s = CH
            while s > 16:
                h = s // 2
                x = x[:h] + x[h:]
                s = h
            acc = x if acc is None else acc + x
        return jnp.sum(acc.astype(jnp.float32), axis=0, keepdims=True)

    def encode(c1, c2):
        c1b = jnp.broadcast_to(c1.astype(jnp.bfloat16), (D, L))
        c2b = jnp.broadcast_to(c2.astype(jnp.bfloat16), (D, L))
        u1 = jax.nn.relu(w1_col * c1b + b1_col)
        u2 = jax.nn.relu(w1_col * c2b + b1_col)
        ut = u1 + u2
        out = jax.lax.dot_general(ut, w2b, (((0,), (0,)), ((), ())),
                                  preferred_element_type=jnp.float32)
        return out + b2x2

    for r in range(BB):
        srow = s_ref[r:r + 1, :]
        drow = d_ref[r:r + 1, :]
        s_corp = jnp.transpose(srow)
        d_corp = jnp.transpose(drow)
        s_q = jnp.where(srow == jnp.int16(0), jnp.int16(-1), srow)
        d_q = jnp.where(drow == jnp.int16(0), jnp.int16(-1), drow)

        c_ss = counts(s_q, s_corp)
        c_sd = counts(s_q, d_corp)
        c_ds = counts(d_q, s_corp)
        c_dd = counts(d_q, d_corp)

        out_s_ref[r] = encode(c_ss, c_sd)
        out_d_ref[r] = encode(c_ds, c_dd)


def kernel(src_ids, dst_ids, W1, b1, W2, b2):
    s16 = src_ids.astype(jnp.int16)
    d16 = dst_ids.astype(jnp.int16)
    w1_col = jnp.broadcast_to(W1.reshape(D, 1), (D, L)).astype(jnp.bfloat16)
    b1_col = jnp.broadcast_to(b1.reshape(D, 1), (D, L)).astype(jnp.bfloat16)
    w2b = W2.astype(jnp.bfloat16)
    b2x2 = (2.0 * b2).reshape(1, D)
    grid = (B // BB,)
    out_shape = jax.ShapeDtypeStruct((B, L, D), jnp.float32)
    src_feat, dst_feat = pl.pallas_call(
        _kernel,
        grid=grid,
        in_specs=[
            pl.BlockSpec((BB, L), lambda i: (i, 0)),
            pl.BlockSpec((BB, L), lambda i: (i, 0)),
            pl.BlockSpec((D, L), lambda i: (0, 0)),
            pl.BlockSpec((D, L), lambda i: (0, 0)),
            pl.BlockSpec((D, D), lambda i: (0, 0)),
            pl.BlockSpec((1, D), lambda i: (0, 0)),
        ],
        out_specs=[
            pl.BlockSpec((BB, L, D), lambda i: (i, 0, 0)),
            pl.BlockSpec((BB, L, D), lambda i: (i, 0, 0)),
        ],
        out_shape=[out_shape, out_shape],
        compiler_params=pltpu.CompilerParams(
            dimension_semantics=("parallel",),
            vmem_limit_bytes=56 * 1024 * 1024,
            fuse_transposed_lhs_in_matmul=True,
        ),
    )(s16, d16, w1_col, b1_col, w2b, b2x2)
    return src_feat, dst_feat


# speedup vs baseline: 1.1725x; 1.1725x over previous
import jax
import jax.numpy as jnp
from jax.experimental import pallas as pl
from jax.experimental.pallas import tpu as pltpu

B, L, D = 256, 512, 128
BB = 16
CH = 128


def _kernel(s_ref, d_ref, w1c_ref, b1c_ref, w2_ref, b2_ref, out_s_ref, out_d_ref):
    w1_col = w1c_ref[...]
    b1_col = b1c_ref[...]
    w2b = w2_ref[...]
    b2x2 = b2_ref[...]

    def counts(query, corpus_col):
        qb = jnp.broadcast_to(query, (CH, L))
        acc = None
        for c in range(0, L, CH):
            corp = jnp.broadcast_to(corpus_col[c:c + CH], (CH, L))
            x = jnp.where(qb == corp, jnp.int16(1), jnp.int16(0))
            s = CH
            while s > 16:
                h = s // 2
                x = x[:h] + x[h:]
                s = h
            acc = x if acc is None else acc + x
        return jnp.sum(acc.astype(jnp.float32), axis=0, keepdims=True)

    def encode(c1, c2):
        c1b = jnp.broadcast_to(c1.astype(jnp.bfloat16), (D, L))
        c2b = jnp.broadcast_to(c2.astype(jnp.bfloat16), (D, L))
        u1 = jax.nn.relu(w1_col * c1b + b1_col)
        u2 = jax.nn.relu(w1_col * c2b + b1_col)
        ut = u1 + u2
        out = jax.lax.dot_general(ut, w2b, (((0,), (0,)), ((), ())),
                                  preferred_element_type=jnp.float32)
        return out + b2x2

    def row_operands(r):
        srow = s_ref[r:r + 1, :]
        drow = d_ref[r:r + 1, :]
        s_corp = jnp.transpose(srow)
        d_corp = jnp.transpose(drow)
        s_q = jnp.where(srow == jnp.int16(0), jnp.int16(-1), srow)
        d_q = jnp.where(drow == jnp.int16(0), jnp.int16(-1), drow)
        return s_q, d_q, s_corp, d_corp

    ops = row_operands(0)
    for r in range(BB):
        s_q, d_q, s_corp, d_corp = ops
        if r + 1 < BB:
            ops = row_operands(r + 1)

        c_ss = counts(s_q, s_corp)
        c_sd = counts(s_q, d_corp)
        c_ds = counts(d_q, s_corp)
        c_dd = counts(d_q, d_corp)

        out_s_ref[r] = encode(c_ss, c_sd)
        out_d_ref[r] = encode(c_ds, c_dd)


def kernel(src_ids, dst_ids, W1, b1, W2, b2):
    s16 = src_ids.astype(jnp.int16)
    d16 = dst_ids.astype(jnp.int16)
    w1_col = jnp.broadcast_to(W1.reshape(D, 1), (D, L)).astype(jnp.bfloat16)
    b1_col = jnp.broadcast_to(b1.reshape(D, 1), (D, L)).astype(jnp.bfloat16)
    w2b = W2.astype(jnp.bfloat16)
    b2x2 = (2.0 * b2).reshape(1, D)
    grid = (B // BB,)
    out_shape = jax.ShapeDtypeStruct((B, L, D), jnp.float32)
    src_feat, dst_feat = pl.pallas_call(
        _kernel,
        grid=grid,
        in_specs=[
            pl.BlockSpec((BB, L), lambda i: (i, 0)),
            pl.BlockSpec((BB, L), lambda i: (i, 0)),
            pl.BlockSpec((D, L), lambda i: (0, 0)),
            pl.BlockSpec((D, L), lambda i: (0, 0)),
            pl.BlockSpec((D, D), lambda i: (0, 0)),
            pl.BlockSpec((1, D), lambda i: (0, 0)),
        ],
        out_specs=[
            pl.BlockSpec((BB, L, D), lambda i: (i, 0, 0)),
            pl.BlockSpec((BB, L, D), lambda i: (i, 0, 0)),
        ],
        out_shape=[out_shape, out_shape],
        compiler_params=pltpu.CompilerParams(
            dimension_semantics=("parallel",),
            vmem_limit_bytes=56 * 1024 * 1024,
            fuse_transposed_lhs_in_matmul=True,
        ),
    )(s16, d16, w1_col, b1_col, w2b, b2x2)
    return src_feat, dst_feat


# speedup vs baseline: 1.1961x; 1.0201x over previous
import jax
import jax.numpy as jnp
from jax.experimental import pallas as pl
from jax.experimental.pallas import tpu as pltpu

B, L, D = 256, 512, 128
BB = 16
CH = 128


def _kernel(s_ref, d_ref, w1c_ref, b1c_ref, w2_ref, b2_ref, out_s_ref, out_d_ref):
    w1_col = w1c_ref[...]
    b1_col = b1c_ref[...]
    w2b = w2_ref[...]
    b2x2 = b2_ref[...]

    def counts(query, corpus_col):
        qb = jnp.broadcast_to(query, (CH, L))
        acc = None
        for c in range(0, L, CH):
            corp = jnp.broadcast_to(corpus_col[c:c + CH], (CH, L))
            x = jnp.where(qb == corp, jnp.int16(1), jnp.int16(0))
            s = CH
            while s > 16:
                h = s // 2
                x = x[:h] + x[h:]
                s = h
            acc = x if acc is None else acc + x
        return jnp.sum(acc.astype(jnp.float32), axis=0, keepdims=True)

    def encode(c1, c2):
        c1b = jnp.broadcast_to(c1.astype(jnp.bfloat16), (D, L))
        c2b = jnp.broadcast_to(c2.astype(jnp.bfloat16), (D, L))
        u1 = jax.nn.relu(w1_col * c1b + b1_col)
        u2 = jax.nn.relu(w1_col * c2b + b1_col)
        ut = u1 + u2
        out = jax.lax.dot_general(ut, w2b, (((0,), (0,)), ((), ())),
                                  preferred_element_type=jnp.float32)
        return out + b2x2

    def row_operands(r):
        srow = s_ref[r:r + 1, :]
        drow = d_ref[r:r + 1, :]
        s_corp = jnp.transpose(srow)
        d_corp = jnp.transpose(drow)
        s_q = jnp.where(srow == jnp.int16(0), jnp.int16(-1), srow)
        d_q = jnp.where(drow == jnp.int16(0), jnp.int16(-1), drow)
        return s_q, d_q, s_corp, d_corp

    LA = 2
    pend = [row_operands(r) for r in range(LA)]
    for r in range(BB):
        s_q, d_q, s_corp, d_corp = pend.pop(0)
        if r + LA < BB:
            pend.append(row_operands(r + LA))

        c_ss = counts(s_q, s_corp)
        c_sd = counts(s_q, d_corp)
        c_ds = counts(d_q, s_corp)
        c_dd = counts(d_q, d_corp)

        out_s_ref[r] = encode(c_ss, c_sd)
        out_d_ref[r] = encode(c_ds, c_dd)


def kernel(src_ids, dst_ids, W1, b1, W2, b2):
    s16 = src_ids.astype(jnp.int16)
    d16 = dst_ids.astype(jnp.int16)
    w1_col = jnp.broadcast_to(W1.reshape(D, 1), (D, L)).astype(jnp.bfloat16)
    b1_col = jnp.broadcast_to(b1.reshape(D, 1), (D, L)).astype(jnp.bfloat16)
    w2b = W2.astype(jnp.bfloat16)
    b2x2 = (2.0 * b2).reshape(1, D)
    grid = (B // BB,)
    out_shape = jax.ShapeDtypeStruct((B, L, D), jnp.float32)
    src_feat, dst_feat = pl.pallas_call(
        _kernel,
        grid=grid,
        in_specs=[
            pl.BlockSpec((BB, L), lambda i: (i, 0)),
            pl.BlockSpec((BB, L), lambda i: (i, 0)),
            pl.BlockSpec((D, L), lambda i: (0, 0)),
            pl.BlockSpec((D, L), lambda i: (0, 0)),
            pl.BlockSpec((D, D), lambda i: (0, 0)),
            pl.BlockSpec((1, D), lambda i: (0, 0)),
        ],
        out_specs=[
            pl.BlockSpec((BB, L, D), lambda i: (i, 0, 0)),
            pl.BlockSpec((BB, L, D), lambda i: (i, 0, 0)),
        ],
        out_shape=[out_shape, out_shape],
        compiler_params=pltpu.CompilerParams(
            dimension_semantics=("parallel",),
            vmem_limit_bytes=56 * 1024 * 1024,
            fuse_transposed_lhs_in_matmul=True,
        ),
    )(s16, d16, w1_col, b1_col, w2b, b2x2)
    return src_feat, dst_feat


# speedup vs baseline: 1.1967x; 1.0005x over previous
import jax
import jax.numpy as jnp
from jax.experimental import pallas as pl
from jax.experimental.pallas import tpu as pltpu

B, L, D = 256, 512, 128
BB = 16
CH = 128


def _kernel(s_ref, d_ref, w1c_ref, b1c_ref, w2_ref, b2_ref, out_s_ref, out_d_ref):
    w1_col = w1c_ref[...]
    b1_col = b1c_ref[...]
    w2b = w2_ref[...]
    b2x2 = b2_ref[...]

    def counts(query, corpus_col):
        qb = jnp.broadcast_to(query, (CH, L))
        acc = None
        for c in range(0, L, CH):
            corp = jnp.broadcast_to(corpus_col[c:c + CH], (CH, L))
            x = jnp.where(qb == corp, jnp.int16(1), jnp.int16(0))
            s = CH
            while s > 16:
                h = s // 2
                x = x[:h] + x[h:]
                s = h
            acc = x if acc is None else acc + x
        return jnp.sum(acc.astype(jnp.float32), axis=0, keepdims=True)

    def encode(c1, c2):
        c1b = jnp.broadcast_to(c1.astype(jnp.bfloat16), (D, L))
        c2b = jnp.broadcast_to(c2.astype(jnp.bfloat16), (D, L))
        u1 = jax.nn.relu(w1_col * c1b + b1_col)
        u2 = jax.nn.relu(w1_col * c2b + b1_col)
        ut = u1 + u2
        out = jax.lax.dot_general(ut, w2b, (((0,), (0,)), ((), ())),
                                  preferred_element_type=jnp.float32)
        return out + b2x2

    def row_operands(r):
        srow = s_ref[r:r + 1, :]
        drow = d_ref[r:r + 1, :]
        s_corp = jnp.transpose(srow)
        d_corp = jnp.transpose(drow)
        s_q = jnp.where(srow == jnp.int16(0), jnp.int16(-1), srow)
        d_q = jnp.where(drow == jnp.int16(0), jnp.int16(-1), drow)
        return s_q, d_q, s_corp, d_corp

    LA = 2
    pend = [row_operands(r) for r in range(LA)]
    for r in range(BB):
        s_q, d_q, s_corp, d_corp = pend.pop(0)
        if r + LA < BB:
            pend.append(row_operands(r + LA))

        c_ss = counts(s_q, s_corp)
        c_sd = counts(s_q, d_corp)
        c_ds = counts(d_q, s_corp)
        c_dd = counts(d_q, d_corp)

        out_s_ref[r] = encode(c_ss, c_sd)
        out_d_ref[r] = encode(c_ds, c_dd)


def kernel(src_ids, dst_ids, W1, b1, W2, b2):
    s16 = src_ids.astype(jnp.int16)
    d16 = dst_ids.astype(jnp.int16)
    w1_col = jnp.broadcast_to(W1.reshape(D, 1), (D, L)).astype(jnp.bfloat16)
    b1_col = jnp.broadcast_to(b1.reshape(D, 1), (D, L)).astype(jnp.bfloat16)
    w2b = W2.astype(jnp.bfloat16)
    b2x2 = (2.0 * b2).reshape(1, D)
    grid = (B // BB,)
    out_shape = jax.ShapeDtypeStruct((B, L, D), jnp.float32)
    src_feat, dst_feat = pl.pallas_call(
        _kernel,
        grid=grid,
        in_specs=[
            pl.BlockSpec((BB, L), lambda i: (i, 0)),
            pl.BlockSpec((BB, L), lambda i: (i, 0)),
            pl.BlockSpec((D, L), lambda i: (0, 0)),
            pl.BlockSpec((D, L), lambda i: (0, 0)),
            pl.BlockSpec((D, D), lambda i: (0, 0)),
            pl.BlockSpec((1, D), lambda i: (0, 0)),
        ],
        out_specs=[
            pl.BlockSpec((BB, L, D), lambda i: (i, 0, 0)),
            pl.BlockSpec((BB, L, D), lambda i: (i, 0, 0)),
        ],
        out_shape=[out_shape, out_shape],
        compiler_params=pltpu.CompilerParams(
            dimension_semantics=("parallel",),
            fuse_transposed_lhs_in_matmul=True,
            vmem_limit_bytes=56 * 1024 * 1024,
        ),
    )(s16, d16, w1_col, b1_col, w2b, b2x2)
    return src_feat, dst_feat
